# baseline (speedup 1.0000x reference)
import functools

import jax
import jax.numpy as jnp
from jax import lax
from jax.experimental import pallas as pl
from jax.experimental.pallas import tpu as pltpu
from jax.experimental.pallas import tpu_sc as plsc

N_NODES = 10000
FEAT = 128
NW = 32
CHUNK = 128
CHUNKS = 84
BLK = 4
NBLOCKS = CHUNKS // BLK
PER_TILE = CHUNKS * CHUNK
TOTAL_SLOTS = NW * PER_TILE
NB = 10240
ROWS_PER_TILE = NB // 16
MM_BLK = 1000
GRID = N_NODES // MM_BLK

def _deg_body(cols_hbm, degp_hbm, col_v, hist_v):
    c = lax.axis_index("c")
    s = lax.axis_index("s")
    w = s * 2 + c
    pltpu.sync_copy(cols_hbm.at[w], col_v)

    def zero(i, carry):
        hist_v[pl.ds(i * 16, 16)] = jnp.zeros((16,), jnp.float32)
        return carry

    lax.fori_loop(0, NB // 16, zero, 0)

    ones = jnp.ones((16,), jnp.float32)

    def body(i, carry):
        j = i // (CHUNK // 16)
        l = i % (CHUNK // 16)
        idx = col_v[j, pl.ds(l * 16, 16)]
        plsc.addupdate_scatter(hist_v, [idx], ones)
        return carry

    lax.fori_loop(0, CHUNKS * (CHUNK // 16), body, 0)
    pltpu.sync_copy(hist_v, degp_hbm.at[w])


def _edge_body(rc_hbm, h_hbm, part_hbm, rci0, rci1, buf0, buf1, acc,
               gs0, gs1, ss0, ss1, is0, is1):
    c = lax.axis_index("c")
    s = lax.axis_index("s")
    w = s * 2 + c
    bufs = (buf0, buf1)
    gsems = (gs0, gs1)
    ssems = (ss0, ss1)

    def zero(i, carry):
        buf0[i // 8, pl.ds((i % 8) * 16, 16)] = jnp.zeros((16,), jnp.float32)
        return carry

    lax.fori_loop(0, CHUNK * 8, zero, 0)
    for t in range(ROWS_PER_TILE // CHUNK):
        pltpu.sync_copy(buf0, acc.at[pl.ds(s * ROWS_PER_TILE + t * CHUNK, CHUNK)])
    plsc.subcore_barrier()

    pltpu.sync_copy(rc_hbm.at[w, pl.ds(0, 2 * BLK)], rci0)
    pltpu.async_copy(rc_hbm.at[w, pl.ds(2 * BLK, 2 * BLK)], rci1, is1)
    pltpu.async_copy(h_hbm.at[rci0.at[0]], buf0, gs0)

    def process_block(b, slot, oslot, isem_next, has_next, first):
        for jj in range(BLK):
            k = jj % 2
            ko = 1 - k
            pltpu.make_async_copy(h_hbm.at[slot.at[2 * jj]], bufs[k], gsems[k]).wait()
            pltpu.async_copy(bufs[k], acc.at[slot.at[2 * jj + 1]], ssems[k], add=True)
            if jj == 0:
                if first:
                    @pl.when(b > 0)
                    def _():
                        pltpu.make_async_copy(
                            bufs[ko], acc.at[oslot.at[2 * BLK - 1]], ssems[ko]).wait()
                else:
                    pltpu.make_async_copy(
                        bufs[ko], acc.at[oslot.at[2 * BLK - 1]], ssems[ko]).wait()
            else:
                pltpu.make_async_copy(
                    bufs[ko], acc.at[slot.at[2 * jj - 1]], ssems[ko]).wait()
            if jj == 1 and has_next and not first:
                pltpu.async_copy(
                    rc_hbm.at[w, pl.ds((b + 1) * 2 * BLK, 2 * BLK)],
                    oslot, isem_next)
            if jj == 1 and has_next and first:
                @pl.when(b > 0)
                def _():
                    pltpu.async_copy(
                        rc_hbm.at[w, pl.ds((b + 1) * 2 * BLK, 2 * BLK)],
                        oslot, isem_next)
            if jj < BLK - 1:
                pltpu.async_copy(h_hbm.at[slot.at[2 * jj + 2]], bufs[ko], gsems[ko])
            elif has_next:
                pltpu.make_async_copy(
                    rc_hbm.at[w, pl.ds((b + 1) * 2 * BLK, 2 * BLK)],
                    oslot, isem_next).wait()
                pltpu.async_copy(h_hbm.at[oslot.at[0]], bufs[ko], gsems[ko])

    def body(q, carry):
        process_block(2 * q, rci0, rci1, is1, True, True)
        process_block(2 * q + 1, rci1, rci0, is0, True, False)
        return carry

    lax.fori_loop(0, (NBLOCKS - 1) // 2, body, 0)
    process_block(NBLOCKS - 1, rci0, rci1, is1, False, False)
    pltpu.make_async_copy(
        bufs[(CHUNKS - 1) % 2], acc.at[rci0.at[2 * BLK - 1]],
        ssems[(CHUNKS - 1) % 2]).wait()
    plsc.subcore_barrier()
    pltpu.sync_copy(
        acc.at[pl.ds(s * ROWS_PER_TILE, ROWS_PER_TILE)],
        part_hbm.at[c].at[pl.ds(s * ROWS_PER_TILE, ROWS_PER_TILE)],
    )


@functools.lru_cache(maxsize=None)
def _sc_calls():
    mesh = plsc.VectorSubcoreMesh(core_axis_name="c", subcore_axis_name="s")
    params = pltpu.CompilerParams(needs_layout_passes=False)
    deg_call = functools.partial(
        pl.kernel,
        out_type=jax.ShapeDtypeStruct((NW, NB), jnp.float32),
        mesh=mesh,
        compiler_params=params,
        scratch_types=[
            pltpu.VMEM((CHUNKS, CHUNK), jnp.int32),
            pltpu.VMEM((NB,), jnp.float32),
        ],
    )(_deg_body)
    edge_call = functools.partial(
        pl.kernel,
        out_type=jax.ShapeDtypeStruct((2, NB, FEAT), jnp.float32),
        mesh=mesh,
        compiler_params=params,
        scratch_types=[
            pltpu.VMEM((2 * BLK, CHUNK), jnp.int32),
            pltpu.VMEM((2 * BLK, CHUNK), jnp.int32),
            pltpu.VMEM((CHUNK, FEAT), jnp.float32),
            pltpu.VMEM((CHUNK, FEAT), jnp.float32),
            pltpu.VMEM_SHARED((NB, FEAT), jnp.float32),
        ] + [pltpu.SemaphoreType.DMA] * 6,
    )(_edge_body)
    return deg_call, edge_call


def _mm_body(x_ref, w_ref, degp_ref, h_ref):
    deg = jnp.sum(degp_ref[...], axis=1)
    dinv = lax.rsqrt(deg)
    h = jnp.dot(x_ref[...], w_ref[...], preferred_element_type=jnp.float32)
    h_ref[...] = h * dinv[:, None]


_mm_call = pl.pallas_call(
    _mm_body,
    grid=(GRID,),
    in_specs=[
        pl.BlockSpec((MM_BLK, FEAT), lambda i: (i, 0)),
        pl.BlockSpec((FEAT, FEAT), lambda i: (0, 0)),
        pl.BlockSpec((MM_BLK, NW), lambda i: (i, 0)),
    ],
    out_specs=pl.BlockSpec((MM_BLK, FEAT), lambda i: (i, 0)),
    out_shape=jax.ShapeDtypeStruct((N_NODES, FEAT), jnp.float32),
)


def _fin_body(p_ref, degp_ref, b_ref, o_ref):
    deg = jnp.sum(degp_ref[...], axis=1)
    dinv = lax.rsqrt(deg)
    ssum = p_ref[0] + p_ref[1]
    o_ref[...] = jnp.maximum(ssum * dinv[:, None] + b_ref[...], 0.0)


_fin_call = pl.pallas_call(
    _fin_body,
    grid=(GRID,),
    in_specs=[
        pl.BlockSpec((2, MM_BLK, FEAT), lambda i: (0, i, 0)),
        pl.BlockSpec((MM_BLK, NW), lambda i: (i, 0)),
        pl.BlockSpec((1, FEAT), lambda i: (0, 0)),
    ],
    out_specs=pl.BlockSpec((MM_BLK, FEAT), lambda i: (i, 0)),
    out_shape=jax.ShapeDtypeStruct((N_NODES, FEAT), jnp.float32),
)


def kernel(x, edge_index, W, b):
    n = x.shape[0]
    loops = jnp.arange(n, dtype=jnp.int32)
    rows_all = jnp.concatenate([edge_index[0].astype(jnp.int32), loops])
    cols_all = jnp.concatenate([edge_index[1].astype(jnp.int32), loops])
    pad = TOTAL_SLOTS - rows_all.shape[0]
    pad_cols = n + (jnp.arange(pad, dtype=jnp.int32) % (NB - n))
    rows_p = jnp.concatenate([rows_all, jnp.zeros((pad,), jnp.int32)])
    cols_p = jnp.concatenate([cols_all, pad_cols])
    rows_p = rows_p.reshape(NW, CHUNKS, CHUNK)
    cols_p = cols_p.reshape(NW, CHUNKS, CHUNK)
    rc = jnp.stack([rows_p, cols_p], axis=2).reshape(NW, 2 * CHUNKS, CHUNK)

    deg_call, edge_call = _sc_calls()
    degp = deg_call(cols_p).T
    hp = _mm_call(x, W, degp)
    part = edge_call(rc, hp)
    return _fin_call(part, degp, b.reshape(1, FEAT))


# speedup vs baseline: 3.9404x; 3.9404x over previous
import functools

import jax
import jax.numpy as jnp
from jax import lax
from jax.experimental import pallas as pl
from jax.experimental.pallas import tpu as pltpu
from jax.experimental.pallas import tpu_sc as plsc

N_NODES = 10000
FEAT = 128
NW = 32
CHUNK = 128
CHUNKS = 84
BLK = 4
NBLOCKS = CHUNKS // BLK
PER_TILE = CHUNKS * CHUNK
TOTAL_SLOTS = NW * PER_TILE
NB = 10240
ROWS_PER_TILE = NB // 16
MM_BLK = 1000
GRID = N_NODES // MM_BLK

def _deg_body(cols_hbm, degp_hbm, col_v, hist_v):
    c = lax.axis_index("c")
    s = lax.axis_index("s")
    w = s * 2 + c
    pltpu.sync_copy(cols_hbm.at[w], col_v)

    def zero(i, carry):
        hist_v[pl.ds(i * 16, 16)] = jnp.zeros((16,), jnp.float32)
        return carry

    lax.fori_loop(0, NB // 16, zero, 0)

    ones = jnp.ones((16,), jnp.float32)

    def body(i, carry):
        j = i // (CHUNK // 16)
        l = i % (CHUNK // 16)
        idx = col_v[j, pl.ds(l * 16, 16)]
        plsc.addupdate_scatter(hist_v, [idx], ones)
        return carry

    lax.fori_loop(0, CHUNKS * (CHUNK // 16), body, 0)
    pltpu.sync_copy(hist_v, degp_hbm.at[w])


def _edge_body(rc_hbm, h_hbm, part_hbm, rci0, rci1, buf0, buf1, acc,
               gs0, gs1, ss0, ss1, is0, is1):
    c = lax.axis_index("c")
    s = lax.axis_index("s")
    w = s * 2 + c
    bufs = (buf0, buf1)
    gsems = (gs0, gs1)
    ssems = (ss0, ss1)

    def zero(i, carry):
        buf0[i // 8, pl.ds((i % 8) * 16, 16)] = jnp.zeros((16,), jnp.float32)
        return carry

    lax.fori_loop(0, CHUNK * 8, zero, 0)
    for t in range(ROWS_PER_TILE // CHUNK):
        pltpu.sync_copy(buf0, acc.at[pl.ds(s * ROWS_PER_TILE + t * CHUNK, CHUNK)])
    plsc.subcore_barrier()

    pltpu.sync_copy(rc_hbm.at[w, pl.ds(0, 2 * BLK)], rci0)
    pltpu.async_copy(rc_hbm.at[w, pl.ds(2 * BLK, 2 * BLK)], rci1, is1)
    pltpu.async_copy(h_hbm.at[rci0.at[0]], buf0, gs0)

    def process_block(b, slot, oslot, isem_next, has_next, first):
        for jj in range(BLK):
            k = jj % 2
            ko = 1 - k
            pltpu.make_async_copy(h_hbm.at[slot.at[2 * jj]], bufs[k], gsems[k]).wait()
            pltpu.async_copy(bufs[k], acc.at[slot.at[2 * jj + 1]], ssems[k], add=True)
            if jj == 0:
                if first:
                    @pl.when(b > 0)
                    def _():
                        pltpu.make_async_copy(
                            bufs[ko], acc.at[oslot.at[2 * BLK - 1]], ssems[ko]).wait()
                else:
                    pltpu.make_async_copy(
                        bufs[ko], acc.at[oslot.at[2 * BLK - 1]], ssems[ko]).wait()
            else:
                pltpu.make_async_copy(
                    bufs[ko], acc.at[slot.at[2 * jj - 1]], ssems[ko]).wait()
            if jj == 1 and has_next and not first:
                pltpu.async_copy(
                    rc_hbm.at[w, pl.ds((b + 1) * 2 * BLK, 2 * BLK)],
                    oslot, isem_next)
            if jj == 1 and has_next and first:
                @pl.when(b > 0)
                def _():
                    pltpu.async_copy(
                        rc_hbm.at[w, pl.ds((b + 1) * 2 * BLK, 2 * BLK)],
                        oslot, isem_next)
            if jj < BLK - 1:
                pltpu.async_copy(h_hbm.at[slot.at[2 * jj + 2]], bufs[ko], gsems[ko])
            elif has_next:
                pltpu.make_async_copy(
                    rc_hbm.at[w, pl.ds((b + 1) * 2 * BLK, 2 * BLK)],
                    oslot, isem_next).wait()
                pltpu.async_copy(h_hbm.at[oslot.at[0]], bufs[ko], gsems[ko])

    def body(q, carry):
        process_block(2 * q, rci0, rci1, is1, True, True)
        process_block(2 * q + 1, rci1, rci0, is0, True, False)
        return carry

    lax.fori_loop(0, (NBLOCKS - 1) // 2, body, 0)
    process_block(NBLOCKS - 1, rci0, rci1, is1, False, False)
    pltpu.make_async_copy(
        bufs[(CHUNKS - 1) % 2], acc.at[rci0.at[2 * BLK - 1]],
        ssems[(CHUNKS - 1) % 2]).wait()
    plsc.subcore_barrier()
    pltpu.sync_copy(
        acc.at[pl.ds(s * ROWS_PER_TILE, ROWS_PER_TILE)],
        part_hbm.at[c].at[pl.ds(s * ROWS_PER_TILE, ROWS_PER_TILE)],
    )


@functools.lru_cache(maxsize=None)
def _sc_calls():
    mesh = plsc.VectorSubcoreMesh(core_axis_name="c", subcore_axis_name="s")
    params = pltpu.CompilerParams(needs_layout_passes=False)
    deg_call = functools.partial(
        pl.kernel,
        out_type=jax.ShapeDtypeStruct((NW, NB), jnp.float32),
        mesh=mesh,
        compiler_params=params,
        scratch_types=[
            pltpu.VMEM((CHUNKS, CHUNK), jnp.int32),
            pltpu.VMEM((NB,), jnp.float32),
        ],
    )(_deg_body)
    edge_call = functools.partial(
        pl.kernel,
        out_type=jax.ShapeDtypeStruct((2, NB, FEAT), jnp.float32),
        mesh=mesh,
        compiler_params=params,
        scratch_types=[
            pltpu.VMEM((2 * BLK, CHUNK), jnp.int32),
            pltpu.VMEM((2 * BLK, CHUNK), jnp.int32),
            pltpu.VMEM((CHUNK, FEAT), jnp.float32),
            pltpu.VMEM((CHUNK, FEAT), jnp.float32),
            pltpu.VMEM_SHARED((NB, FEAT), jnp.float32),
        ] + [pltpu.SemaphoreType.DMA] * 6,
    )(_edge_body)
    return deg_call, edge_call


def _mm_body(x_ref, w_ref, degp_ref, h_ref):
    deg = jnp.sum(degp_ref[...], axis=1)
    dinv = lax.rsqrt(deg)
    h = jnp.dot(x_ref[...], w_ref[...], preferred_element_type=jnp.float32)
    h_ref[...] = h * dinv[:, None]


_mm_call = pl.pallas_call(
    _mm_body,
    grid=(GRID,),
    in_specs=[
        pl.BlockSpec((MM_BLK, FEAT), lambda i: (i, 0)),
        pl.BlockSpec((FEAT, FEAT), lambda i: (0, 0)),
        pl.BlockSpec((MM_BLK, NW), lambda i: (i, 0)),
    ],
    out_specs=pl.BlockSpec((MM_BLK, FEAT), lambda i: (i, 0)),
    out_shape=jax.ShapeDtypeStruct((N_NODES, FEAT), jnp.float32),
)


def _fin_body(p_ref, degp_ref, b_ref, o_ref):
    deg = jnp.sum(degp_ref[...], axis=1)
    dinv = lax.rsqrt(deg)
    ssum = p_ref[0] + p_ref[1]
    o_ref[...] = jnp.maximum(ssum * dinv[:, None] + b_ref[...], 0.0)


_fin_call = pl.pallas_call(
    _fin_body,
    grid=(GRID,),
    in_specs=[
        pl.BlockSpec((2, MM_BLK, FEAT), lambda i: (0, i, 0)),
        pl.BlockSpec((MM_BLK, NW), lambda i: (i, 0)),
        pl.BlockSpec((1, FEAT), lambda i: (0, 0)),
    ],
    out_specs=pl.BlockSpec((MM_BLK, FEAT), lambda i: (i, 0)),
    out_shape=jax.ShapeDtypeStruct((N_NODES, FEAT), jnp.float32),
)


def kernel(x, edge_index, W, b):
    n = x.shape[0]
    loops = jnp.arange(n, dtype=jnp.int32)
    rows_all = jnp.concatenate([edge_index[0].astype(jnp.int32), loops])
    cols_all = jnp.concatenate([edge_index[1].astype(jnp.int32), loops])
    pad = TOTAL_SLOTS - rows_all.shape[0]
    pad_iota = jnp.arange(pad, dtype=jnp.int32)
    pad_cols = n + pad_iota % (NB - n)
    rows_p = jnp.concatenate([rows_all, pad_iota * 61 % n])
    cols_p = jnp.concatenate([cols_all, pad_cols])
    rows_p = rows_p.reshape(NW, CHUNKS, CHUNK)
    cols_p = cols_p.reshape(NW, CHUNKS, CHUNK)
    rc = jnp.stack([rows_p, cols_p], axis=2).reshape(NW, 2 * CHUNKS, CHUNK)

    deg_call, edge_call = _sc_calls()
    degp = deg_call(cols_p).T
    hp = _mm_call(x, W, degp)
    part = edge_call(rc, hp)
    return _fin_call(part, degp, b.reshape(1, FEAT))


# speedup vs baseline: 3.9950x; 1.0139x over previous
import functools

import jax
import jax.numpy as jnp
from jax import lax
from jax.experimental import pallas as pl
from jax.experimental.pallas import tpu as pltpu
from jax.experimental.pallas import tpu_sc as plsc

N_NODES = 10000
FEAT = 128
NW = 32
CHUNK = 128
CHUNKS = 84
BLK = 4
NBLOCKS = CHUNKS // BLK
PER_TILE = CHUNKS * CHUNK
TOTAL_SLOTS = NW * PER_TILE
NB = 10240
ROWS_PER_TILE = NB // 16
MM_BLK = 1000
GRID = N_NODES // MM_BLK

def _deg_body(cols_hbm, degp_hbm, col_v, hist_v):
    c = lax.axis_index("c")
    s = lax.axis_index("s")
    w = s * 2 + c
    pltpu.sync_copy(cols_hbm.at[w], col_v)

    def zero(i, carry):
        hist_v[pl.ds(i * 16, 16)] = jnp.zeros((16,), jnp.float32)
        return carry

    lax.fori_loop(0, NB // 16, zero, 0)

    ones = jnp.ones((16,), jnp.float32)

    def body(i, carry):
        j = i // (CHUNK // 16)
        l = i % (CHUNK // 16)
        idx = col_v[j, pl.ds(l * 16, 16)]
        plsc.addupdate_scatter(hist_v, [idx], ones)
        return carry

    lax.fori_loop(0, CHUNKS * (CHUNK // 16), body, 0)
    pltpu.sync_copy(hist_v, degp_hbm.at[w])


def _edge_body(rc_hbm, h_hbm, part_hbm, rci0, rci1, buf0, buf1, acc,
               ga0, ga1, gb0, gb1, ss0, ss1, is0, is1):
    c = lax.axis_index("c")
    s = lax.axis_index("s")
    w = s * 2 + c
    bufs = (buf0, buf1)
    gasems = (ga0, ga1)
    gbsems = (gb0, gb1)
    ssems = (ss0, ss1)
    half = CHUNK // 2

    def gstart(slot, row, k):
        pltpu.async_copy(h_hbm.at[slot.at[row, pl.ds(0, half)]],
                         bufs[k].at[pl.ds(0, half)], gasems[k])
        pltpu.async_copy(h_hbm.at[slot.at[row, pl.ds(half, half)]],
                         bufs[k].at[pl.ds(half, half)], gbsems[k])

    def gwait(slot, row, k):
        pltpu.make_async_copy(h_hbm.at[slot.at[row, pl.ds(0, half)]],
                              bufs[k].at[pl.ds(0, half)], gasems[k]).wait()
        pltpu.make_async_copy(h_hbm.at[slot.at[row, pl.ds(half, half)]],
                              bufs[k].at[pl.ds(half, half)], gbsems[k]).wait()

    def zero(i, carry):
        buf0[i // 8, pl.ds((i % 8) * 16, 16)] = jnp.zeros((16,), jnp.float32)
        return carry

    lax.fori_loop(0, CHUNK * 8, zero, 0)
    for t in range(ROWS_PER_TILE // CHUNK):
        pltpu.sync_copy(buf0, acc.at[pl.ds(s * ROWS_PER_TILE + t * CHUNK, CHUNK)])
    plsc.subcore_barrier()

    pltpu.sync_copy(rc_hbm.at[w, pl.ds(0, 2 * BLK)], rci0)
    pltpu.async_copy(rc_hbm.at[w, pl.ds(2 * BLK, 2 * BLK)], rci1, is1)
    gstart(rci0, 0, 0)

    def process_block(b, slot, oslot, isem_next, has_next, first):
        for jj in range(BLK):
            k = jj % 2
            ko = 1 - k
            gwait(slot, 2 * jj, k)
            pltpu.async_copy(bufs[k], acc.at[slot.at[2 * jj + 1]], ssems[k], add=True)
            if jj == 0:
                if first:
                    @pl.when(b > 0)
                    def _():
                        pltpu.make_async_copy(
                            bufs[ko], acc.at[oslot.at[2 * BLK - 1]], ssems[ko]).wait()
                else:
                    pltpu.make_async_copy(
                        bufs[ko], acc.at[oslot.at[2 * BLK - 1]], ssems[ko]).wait()
            else:
                pltpu.make_async_copy(
                    bufs[ko], acc.at[slot.at[2 * jj - 1]], ssems[ko]).wait()
            if jj == 1 and has_next and not first:
                pltpu.async_copy(
                    rc_hbm.at[w, pl.ds((b + 1) * 2 * BLK, 2 * BLK)],
                    oslot, isem_next)
            if jj == 1 and has_next and first:
                @pl.when(b > 0)
                def _():
                    pltpu.async_copy(
                        rc_hbm.at[w, pl.ds((b + 1) * 2 * BLK, 2 * BLK)],
                        oslot, isem_next)
            if jj < BLK - 1:
                gstart(slot, 2 * jj + 2, ko)
            elif has_next:
                pltpu.make_async_copy(
                    rc_hbm.at[w, pl.ds((b + 1) * 2 * BLK, 2 * BLK)],
                    oslot, isem_next).wait()
                gstart(oslot, 0, ko)

    def body(q, carry):
        process_block(2 * q, rci0, rci1, is1, True, True)
        process_block(2 * q + 1, rci1, rci0, is0, True, False)
        return carry

    lax.fori_loop(0, (NBLOCKS - 1) // 2, body, 0)
    process_block(NBLOCKS - 1, rci0, rci1, is1, False, False)
    pltpu.make_async_copy(
        bufs[(CHUNKS - 1) % 2], acc.at[rci0.at[2 * BLK - 1]],
        ssems[(CHUNKS - 1) % 2]).wait()
    plsc.subcore_barrier()
    pltpu.sync_copy(
        acc.at[pl.ds(s * ROWS_PER_TILE, ROWS_PER_TILE)],
        part_hbm.at[c].at[pl.ds(s * ROWS_PER_TILE, ROWS_PER_TILE)],
    )


@functools.lru_cache(maxsize=None)
def _sc_calls():
    mesh = plsc.VectorSubcoreMesh(core_axis_name="c", subcore_axis_name="s")
    params = pltpu.CompilerParams(needs_layout_passes=False)
    deg_call = functools.partial(
        pl.kernel,
        out_type=jax.ShapeDtypeStruct((NW, NB), jnp.float32),
        mesh=mesh,
        compiler_params=params,
        scratch_types=[
            pltpu.VMEM((CHUNKS, CHUNK), jnp.int32),
            pltpu.VMEM((NB,), jnp.float32),
        ],
    )(_deg_body)
    edge_call = functools.partial(
        pl.kernel,
        out_type=jax.ShapeDtypeStruct((2, NB, FEAT), jnp.float32),
        mesh=mesh,
        compiler_params=params,
        scratch_types=[
            pltpu.VMEM((2 * BLK, CHUNK), jnp.int32),
            pltpu.VMEM((2 * BLK, CHUNK), jnp.int32),
            pltpu.VMEM((CHUNK, FEAT), jnp.float32),
            pltpu.VMEM((CHUNK, FEAT), jnp.float32),
            pltpu.VMEM_SHARED((NB, FEAT), jnp.float32),
        ] + [pltpu.SemaphoreType.DMA] * 8,
    )(_edge_body)
    return deg_call, edge_call


def _mm_body(x_ref, w_ref, degp_ref, h_ref):
    deg = jnp.sum(degp_ref[...], axis=1)
    dinv = lax.rsqrt(deg)
    h = jnp.dot(x_ref[...], w_ref[...], preferred_element_type=jnp.float32)
    h_ref[...] = h * dinv[:, None]


_mm_call = pl.pallas_call(
    _mm_body,
    grid=(GRID,),
    in_specs=[
        pl.BlockSpec((MM_BLK, FEAT), lambda i: (i, 0)),
        pl.BlockSpec((FEAT, FEAT), lambda i: (0, 0)),
        pl.BlockSpec((MM_BLK, NW), lambda i: (i, 0)),
    ],
    out_specs=pl.BlockSpec((MM_BLK, FEAT), lambda i: (i, 0)),
    out_shape=jax.ShapeDtypeStruct((N_NODES, FEAT), jnp.float32),
)


def _fin_body(p_ref, degp_ref, b_ref, o_ref):
    deg = jnp.sum(degp_ref[...], axis=1)
    dinv = lax.rsqrt(deg)
    ssum = p_ref[0] + p_ref[1]
    o_ref[...] = jnp.maximum(ssum * dinv[:, None] + b_ref[...], 0.0)


_fin_call = pl.pallas_call(
    _fin_body,
    grid=(GRID,),
    in_specs=[
        pl.BlockSpec((2, MM_BLK, FEAT), lambda i: (0, i, 0)),
        pl.BlockSpec((MM_BLK, NW), lambda i: (i, 0)),
        pl.BlockSpec((1, FEAT), lambda i: (0, 0)),
    ],
    out_specs=pl.BlockSpec((MM_BLK, FEAT), lambda i: (i, 0)),
    out_shape=jax.ShapeDtypeStruct((N_NODES, FEAT), jnp.float32),
)


def kernel(x, edge_index, W, b):
    n = x.shape[0]
    loops = jnp.arange(n, dtype=jnp.int32)
    rows_all = jnp.concatenate([edge_index[0].astype(jnp.int32), loops])
    cols_all = jnp.concatenate([edge_index[1].astype(jnp.int32), loops])
    pad = TOTAL_SLOTS - rows_all.shape[0]
    pad_iota = jnp.arange(pad, dtype=jnp.int32)
    pad_cols = n + pad_iota % (NB - n)
    rows_p = jnp.concatenate([rows_all, pad_iota * 61 % n])
    cols_p = jnp.concatenate([cols_all, pad_cols])
    rows_p = rows_p.reshape(NW, CHUNKS, CHUNK)
    cols_p = cols_p.reshape(NW, CHUNKS, CHUNK)
    rc = jnp.stack([rows_p, cols_p], axis=2).reshape(NW, 2 * CHUNKS, CHUNK)

    deg_call, edge_call = _sc_calls()
    degp = deg_call(cols_p).T
    hp = _mm_call(x, W, degp)
    part = edge_call(rc, hp)
    return _fin_call(part, degp, b.reshape(1, FEAT))


# speedup vs baseline: 4.5332x; 1.1347x over previous
import functools

import jax
import jax.numpy as jnp
from jax import lax
from jax.experimental import pallas as pl
from jax.experimental.pallas import tpu as pltpu
from jax.experimental.pallas import tpu_sc as plsc

N_NODES = 10000
FEAT = 128
NW = 32
CHUNK = 128
CHUNKS = 84
BLK = 4
NBLOCKS = CHUNKS // BLK
PER_TILE = CHUNKS * CHUNK
TOTAL_SLOTS = NW * PER_TILE
NB = 10240
ROWS_PER_TILE = NB // 16
MM_BLK = 1000
GRID = N_NODES // MM_BLK

def _deg_body(cols_hbm, degp_hbm, col_v, hist_v):
    c = lax.axis_index("c")
    s = lax.axis_index("s")
    w = s * 2 + c
    pltpu.sync_copy(cols_hbm.at[w], col_v)

    def zero(i, carry):
        hist_v[pl.ds(i * 16, 16)] = jnp.zeros((16,), jnp.float32)
        return carry

    lax.fori_loop(0, NB // 16, zero, 0)

    ones = jnp.ones((16,), jnp.float32)

    def body(i, carry):
        j = i // (CHUNK // 16)
        l = i % (CHUNK // 16)
        idx = col_v[j, pl.ds(l * 16, 16)]
        plsc.addupdate_scatter(hist_v, [idx], ones)
        return carry

    lax.fori_loop(0, CHUNKS * (CHUNK // 16), body, 0)
    pltpu.sync_copy(hist_v, degp_hbm.at[w])


def _edge_body(rc_hbm, h_hbm, part_hbm, rci0, rci1, b0, b1, b2, b3, acc,
               gs0, gs1, gs2, gs3, ss0, ss1, ss2, ss3, is0, is1):
    c = lax.axis_index("c")
    s = lax.axis_index("s")
    w = s * 2 + c
    bufs = (b0, b1, b2, b3)
    gsems = (gs0, gs1, gs2, gs3)
    ssems = (ss0, ss1, ss2, ss3)
    half = CHUNK // 2

    def ridx(slot, t):
        jj, h = t // 2, t % 2
        return slot.at[2 * jj, pl.ds(half * h, half)]

    def cidx(slot, t):
        jj, h = t // 2, t % 2
        return slot.at[2 * jj + 1, pl.ds(half * h, half)]

    def zero(i, carry):
        b0[i // 8, pl.ds((i % 8) * 16, 16)] = jnp.zeros((16,), jnp.float32)
        return carry

    lax.fori_loop(0, half * 8, zero, 0)
    for t in range(ROWS_PER_TILE // half):
        pltpu.sync_copy(b0, acc.at[pl.ds(s * ROWS_PER_TILE + t * half, half)])
    plsc.subcore_barrier()

    pltpu.sync_copy(rc_hbm.at[w, pl.ds(0, 2 * BLK)], rci0)
    pltpu.async_copy(rc_hbm.at[w, pl.ds(2 * BLK, 2 * BLK)], rci1, is1)
    for t in range(3):
        pltpu.async_copy(h_hbm.at[ridx(rci0, t)], bufs[t], gsems[t])

    SUB = 2 * BLK

    def process_block(b, slot, oslot, isem_next, has_next, first):
        for t in range(SUB):
            k = t % 4
            km = (k + 3) % 4
            pltpu.make_async_copy(h_hbm.at[ridx(slot, t)], bufs[k], gsems[k]).wait()
            pltpu.async_copy(bufs[k], acc.at[cidx(slot, t)], ssems[k], add=True)
            if t == 0:
                if first:
                    @pl.when(b > 0)
                    def _():
                        pltpu.make_async_copy(
                            bufs[km], acc.at[cidx(oslot, SUB - 1)], ssems[km]).wait()
                else:
                    pltpu.make_async_copy(
                        bufs[km], acc.at[cidx(oslot, SUB - 1)], ssems[km]).wait()
            else:
                pltpu.make_async_copy(
                    bufs[km], acc.at[cidx(slot, t - 1)], ssems[km]).wait()
            if t == 1 and has_next and not first:
                pltpu.async_copy(
                    rc_hbm.at[w, pl.ds((b + 1) * 2 * BLK, 2 * BLK)],
                    oslot, isem_next)
            if t == 1 and has_next and first:
                @pl.when(b > 0)
                def _():
                    pltpu.async_copy(
                        rc_hbm.at[w, pl.ds((b + 1) * 2 * BLK, 2 * BLK)],
                        oslot, isem_next)
            if t <= SUB - 4:
                pltpu.async_copy(h_hbm.at[ridx(slot, t + 3)], bufs[km], gsems[km])
            elif has_next:
                if t == SUB - 3:
                    pltpu.make_async_copy(
                        rc_hbm.at[w, pl.ds((b + 1) * 2 * BLK, 2 * BLK)],
                        oslot, isem_next).wait()
                pltpu.async_copy(
                    h_hbm.at[ridx(oslot, t - (SUB - 3))], bufs[km], gsems[km])

    def body(q, carry):
        process_block(2 * q, rci0, rci1, is1, True, True)
        process_block(2 * q + 1, rci1, rci0, is0, True, False)
        return carry

    lax.fori_loop(0, (NBLOCKS - 1) // 2, body, 0)
    process_block(NBLOCKS - 1, rci0, rci1, is1, False, False)
    pltpu.make_async_copy(
        bufs[(2 * CHUNKS - 1) % 4], acc.at[cidx(rci0, SUB - 1)],
        ssems[(2 * CHUNKS - 1) % 4]).wait()
    plsc.subcore_barrier()
    pltpu.sync_copy(
        acc.at[pl.ds(s * ROWS_PER_TILE, ROWS_PER_TILE)],
        part_hbm.at[c].at[pl.ds(s * ROWS_PER_TILE, ROWS_PER_TILE)],
    )


@functools.lru_cache(maxsize=None)
def _sc_calls():
    mesh = plsc.VectorSubcoreMesh(core_axis_name="c", subcore_axis_name="s")
    params = pltpu.CompilerParams(needs_layout_passes=False)
    deg_call = functools.partial(
        pl.kernel,
        out_type=jax.ShapeDtypeStruct((NW, NB), jnp.float32),
        mesh=mesh,
        compiler_params=params,
        scratch_types=[
            pltpu.VMEM((CHUNKS, CHUNK), jnp.int32),
            pltpu.VMEM((NB,), jnp.float32),
        ],
    )(_deg_body)
    edge_call = functools.partial(
        pl.kernel,
        out_type=jax.ShapeDtypeStruct((2, NB, FEAT), jnp.float32),
        mesh=mesh,
        compiler_params=params,
        scratch_types=[
            pltpu.VMEM((2 * BLK, CHUNK), jnp.int32),
            pltpu.VMEM((2 * BLK, CHUNK), jnp.int32),
            pltpu.VMEM((CHUNK // 2, FEAT), jnp.float32),
            pltpu.VMEM((CHUNK // 2, FEAT), jnp.float32),
            pltpu.VMEM((CHUNK // 2, FEAT), jnp.float32),
            pltpu.VMEM((CHUNK // 2, FEAT), jnp.float32),
            pltpu.VMEM_SHARED((NB, FEAT), jnp.float32),
        ] + [pltpu.SemaphoreType.DMA] * 10,
    )(_edge_body)
    return deg_call, edge_call


def _mm_body(x_ref, w_ref, degp_ref, h_ref):
    deg = jnp.sum(degp_ref[...], axis=1)
    dinv = lax.rsqrt(deg)
    h = jnp.dot(x_ref[...], w_ref[...], preferred_element_type=jnp.float32)
    h_ref[...] = h * dinv[:, None]


_mm_call = pl.pallas_call(
    _mm_body,
    grid=(GRID,),
    in_specs=[
        pl.BlockSpec((MM_BLK, FEAT), lambda i: (i, 0)),
        pl.BlockSpec((FEAT, FEAT), lambda i: (0, 0)),
        pl.BlockSpec((MM_BLK, NW), lambda i: (i, 0)),
    ],
    out_specs=pl.BlockSpec((MM_BLK, FEAT), lambda i: (i, 0)),
    out_shape=jax.ShapeDtypeStruct((N_NODES, FEAT), jnp.float32),
)


def _fin_body(p_ref, degp_ref, b_ref, o_ref):
    deg = jnp.sum(degp_ref[...], axis=1)
    dinv = lax.rsqrt(deg)
    ssum = p_ref[0] + p_ref[1]
    o_ref[...] = jnp.maximum(ssum * dinv[:, None] + b_ref[...], 0.0)


_fin_call = pl.pallas_call(
    _fin_body,
    grid=(GRID,),
    in_specs=[
        pl.BlockSpec((2, MM_BLK, FEAT), lambda i: (0, i, 0)),
        pl.BlockSpec((MM_BLK, NW), lambda i: (i, 0)),
        pl.BlockSpec((1, FEAT), lambda i: (0, 0)),
    ],
    out_specs=pl.BlockSpec((MM_BLK, FEAT), lambda i: (i, 0)),
    out_shape=jax.ShapeDtypeStruct((N_NODES, FEAT), jnp.float32),
)


def kernel(x, edge_index, W, b):
    n = x.shape[0]
    loops = jnp.arange(n, dtype=jnp.int32)
    rows_all = jnp.concatenate([edge_index[0].astype(jnp.int32), loops])
    cols_all = jnp.concatenate([edge_index[1].astype(jnp.int32), loops])
    pad = TOTAL_SLOTS - rows_all.shape[0]
    pad_iota = jnp.arange(pad, dtype=jnp.int32)
    pad_cols = n + pad_iota % (NB - n)
    rows_p = jnp.concatenate([rows_all, pad_iota * 61 % n])
    cols_p = jnp.concatenate([cols_all, pad_cols])
    rows_p = rows_p.reshape(NW, CHUNKS, CHUNK)
    cols_p = cols_p.reshape(NW, CHUNKS, CHUNK)
    rc = jnp.stack([rows_p, cols_p], axis=2).reshape(NW, 2 * CHUNKS, CHUNK)

    deg_call, edge_call = _sc_calls()
    degp = deg_call(cols_p).T
    hp = _mm_call(x, W, degp)
    part = edge_call(rc, hp)
    return _fin_call(part, degp, b.reshape(1, FEAT))
